# baseline (speedup 1.0000x reference)
import functools

import jax
import jax.numpy as jnp
from jax import lax
from jax.experimental import pallas as pl
from jax.experimental.pallas import tpu as pltpu
from jax.experimental.pallas import tpu_sc as plsc

N_NODES = 10000
D = 128
E = 320000
NUM_GROUPS = 4
BN_EPS = 1e-5

NC = 2
NS = 16
NW = NC * NS

CHUNK = 80
NBUF = 4
WCHUNKS = 8
NPAIR = 8
CHUNKS_PER_TILE = 2 * WCHUNKS * NPAIR
EDGES_PER_TILE = CHUNK * CHUNKS_PER_TILE
E_PAD = EDGES_PER_TILE * NW
AGG_ROWS = 10112
ROWS_PER_TILE = AGG_ROWS // NS

_mesh = plsc.VectorSubcoreMesh(core_axis_name="c", subcore_axis_name="s")


@functools.partial(
    pl.kernel,
    out_type=jax.ShapeDtypeStruct((NC, AGG_ROWS, D), jnp.float32),
    mesh=_mesh,
    scratch_types=[
        pltpu.VMEM((2, 2, WCHUNKS, CHUNK), jnp.int32),
        pltpu.VMEM((NBUF, CHUNK, D), jnp.float32),
        pltpu.VMEM_SHARED((AGG_ROWS, D), jnp.float32),
        pltpu.SemaphoreType.DMA,
        pltpu.SemaphoreType.DMA,
        pltpu.SemaphoreType.DMA,
        pltpu.SemaphoreType.DMA,
        pltpu.SemaphoreType.DMA,
    ],
)
def _sc_agg(data_hbm, zeros_hbm, idx_hbm, out_hbm,
            idx_w, rows_v, agg_sh, sem0, sem1, sem2, sem3, sem_i):
    cid = lax.axis_index("c")
    sid = lax.axis_index("s")
    wid = sid * NC + cid
    r0 = sid * ROWS_PER_TILE
    sems = (sem0, sem1, sem2, sem3)

    pltpu.sync_copy(zeros_hbm.at[pl.ds(r0, ROWS_PER_TILE)],
                    agg_sh.at[pl.ds(r0, ROWS_PER_TILE)])

    def _load_window(w, half, sync):
        cp = pltpu.async_copy(idx_hbm.at[wid, w], idx_w.at[half], sem_i)
        if sync:
            cp.wait()

    def _wait_window():
        pltpu.make_async_copy(idx_hbm.at[0, 0], idx_w.at[0], sem_i).wait()

    def _gather(half, pos, b):
        pltpu.async_copy(data_hbm.at[idx_w.at[half, 0, pos]],
                         rows_v.at[b], sems[b])

    def _wait_gather(b):
        pltpu.make_async_copy(data_hbm.at[pl.ds(0, CHUNK)], rows_v.at[b],
                              sems[b]).wait()

    def _scatter(half, pos, b):
        pltpu.sync_copy(rows_v.at[b], agg_sh.at[idx_w.at[half, 1, pos]],
                        add=True)

    _load_window(0, 0, True)
    plsc.subcore_barrier()
    _load_window(1, 1, False)
    for jj in range(NBUF):
        _gather(0, jj, jj)

    def body(p, carry):
        not_last = p < NPAIR - 1
        for jj in range(16):
            half, pos, b = (jj // 8) % 2, jj % 8, jj % NBUF
            nhalf, npos = ((jj + 4) // 8) % 2, (jj + 4) % 8
            if jj == 4:
                _wait_window()
            if jj == 12:

                @pl.when(not_last)
                def _():
                    _wait_window()

            if jj < 12:
                _wait_gather(b)
                _scatter(half, pos, b)
                _gather(nhalf, npos, b)
            else:
                @pl.when(not_last)
                def _():
                    _wait_gather(b)
                    _scatter(half, pos, b)
                    _gather(nhalf, npos, b)

            if jj == 7:

                @pl.when(not_last)
                def _():
                    _load_window(2 * p + 2, 0, False)

            if jj == 15:

                @pl.when(not_last)
                def _():
                    _load_window(2 * p + 3, 1, False)
        return carry

    lax.fori_loop(0, NPAIR, body, 0)

    for jj in range(12, 16):
        b = jj % NBUF
        _wait_gather(b)
        _scatter(1, jj % 8, b)

    plsc.subcore_barrier()
    pltpu.sync_copy(agg_sh.at[pl.ds(r0, ROWS_PER_TILE)],
                    out_hbm.at[cid, pl.ds(r0, ROWS_PER_TILE)])


def _tc_body(data_ref, p0_ref, p1_ref, wr_ref, wn_ref, b_ref, gam_ref,
             bet_ref, g_ref, gt_ref, out_ref):
    x = data_ref[...]
    agg = p0_ref[...] + p1_ref[...]
    acc = jnp.dot(x, wr_ref[...], preferred_element_type=jnp.float32)
    acc = acc + jnp.dot(agg, wn_ref[...], preferred_element_type=jnp.float32)
    acc = acc + b_ref[...]
    g_mat = g_ref[...]
    gt_mat = gt_ref[...]
    m = jnp.dot(jnp.dot(acc, g_mat, preferred_element_type=jnp.float32),
                gt_mat, preferred_element_type=jnp.float32)
    e2 = jnp.dot(jnp.dot(acc * acc, g_mat, preferred_element_type=jnp.float32),
                 gt_mat, preferred_element_type=jnp.float32)
    var = e2 - m * m
    inv = lax.rsqrt(var + BN_EPS)
    out_ref[...] = (acc - m) * inv * gam_ref[...] + bet_ref[...]


def kernel(data, edge_index, depth, W_root, W_neigh, b, gamma, beta):
    del depth
    src = edge_index[0].astype(jnp.int32)
    dst = edge_index[1].astype(jnp.int32)
    pad = E_PAD - E
    pad_ar = jnp.arange(pad, dtype=jnp.int32)
    src_p = jnp.concatenate([src, pad_ar % N_NODES])
    src_p = src_p.reshape(NW, 2 * NPAIR, WCHUNKS, CHUNK)
    dst_p = jnp.concatenate([dst, N_NODES + pad_ar % (AGG_ROWS - N_NODES)])
    dst_p = dst_p.reshape(NW, 2 * NPAIR, WCHUNKS, CHUNK)
    idx = jnp.stack([src_p, dst_p], axis=2)
    zeros = jnp.zeros((AGG_ROWS, D), jnp.float32)

    partials = _sc_agg(data, zeros, idx)

    gsz = D // NUM_GROUPS
    ch = jnp.arange(D, dtype=jnp.int32) // gsz
    gr = jnp.arange(8, dtype=jnp.int32)
    g_mat = (ch[:, None] == gr[None, :]).astype(jnp.float32) / gsz
    gt_mat = (gr[:, None] == ch[None, :]).astype(jnp.float32)

    blk = 1000
    grid = (N_NODES // blk,)
    row_spec = pl.BlockSpec((blk, D), lambda i: (i, 0))
    full = lambda r, c: pl.BlockSpec((r, c), lambda i: (0, 0))
    out = pl.pallas_call(
        _tc_body,
        grid=grid,
        in_specs=[
            row_spec,
            row_spec,
            row_spec,
            full(D, D),
            full(D, D),
            full(1, D),
            full(1, D),
            full(1, D),
            full(D, 8),
            full(8, D),
        ],
        out_specs=row_spec,
        out_shape=jax.ShapeDtypeStruct((N_NODES, D), jnp.float32),
    )(data, partials[0], partials[1], W_root, W_neigh,
      b.reshape(1, D), gamma.reshape(1, D), beta.reshape(1, D), g_mat, gt_mat)
    return out


# speedup vs baseline: 1.0053x; 1.0053x over previous
import functools

import jax
import jax.numpy as jnp
from jax import lax
from jax.experimental import pallas as pl
from jax.experimental.pallas import tpu as pltpu
from jax.experimental.pallas import tpu_sc as plsc

N_NODES = 10000
D = 128
E = 320000
NUM_GROUPS = 4
BN_EPS = 1e-5

NC = 2
NS = 16
NW = NC * NS

CHUNK = 80
NBUF = 4
WCHUNKS = 8
NPAIR = 8
CHUNKS_PER_TILE = 2 * WCHUNKS * NPAIR
EDGES_PER_TILE = CHUNK * CHUNKS_PER_TILE
E_PAD = EDGES_PER_TILE * NW
AGG_ROWS = 10112
ROWS_PER_TILE = AGG_ROWS // NS

_mesh = plsc.VectorSubcoreMesh(core_axis_name="c", subcore_axis_name="s")


@functools.partial(
    pl.kernel,
    out_type=jax.ShapeDtypeStruct((NC, AGG_ROWS, D), jnp.float32),
    mesh=_mesh,
    scratch_types=[
        pltpu.VMEM((2, 2, WCHUNKS, CHUNK), jnp.int32),
        pltpu.VMEM((NBUF, CHUNK, D), jnp.float32),
        pltpu.VMEM_SHARED((AGG_ROWS, D), jnp.float32),
        pltpu.SemaphoreType.DMA,
        pltpu.SemaphoreType.DMA,
        pltpu.SemaphoreType.DMA,
        pltpu.SemaphoreType.DMA,
        pltpu.SemaphoreType.DMA,
    ],
)
def _sc_agg(data_hbm, zeros_hbm, src_hbm, dst_hbm, out_hbm,
            idx_w, rows_v, agg_sh, sem0, sem1, sem2, sem3, sem_i):
    cid = lax.axis_index("c")
    sid = lax.axis_index("s")
    wid = sid * NC + cid
    r0 = sid * ROWS_PER_TILE
    sems = (sem0, sem1, sem2, sem3)

    pltpu.sync_copy(zeros_hbm.at[pl.ds(r0, ROWS_PER_TILE)],
                    agg_sh.at[pl.ds(r0, ROWS_PER_TILE)])

    def _load_window(w, half, sync):
        pltpu.async_copy(src_hbm.at[wid, w], idx_w.at[half, 0], sem_i)
        cp = pltpu.async_copy(dst_hbm.at[wid, w], idx_w.at[half, 1], sem_i)
        if sync:
            _wait_window()

    def _wait_window():
        pltpu.make_async_copy(src_hbm.at[0, 0], idx_w.at[0, 0], sem_i).wait()
        pltpu.make_async_copy(dst_hbm.at[0, 0], idx_w.at[0, 1], sem_i).wait()

    def _gather(half, pos, b):
        pltpu.async_copy(data_hbm.at[idx_w.at[half, 0, pos]],
                         rows_v.at[b], sems[b])

    def _wait_gather(b):
        pltpu.make_async_copy(data_hbm.at[pl.ds(0, CHUNK)], rows_v.at[b],
                              sems[b]).wait()

    def _scatter(half, pos, b):
        pltpu.sync_copy(rows_v.at[b], agg_sh.at[idx_w.at[half, 1, pos]],
                        add=True)

    _load_window(0, 0, True)
    plsc.subcore_barrier()
    _load_window(1, 1, False)
    for jj in range(NBUF):
        _gather(0, jj, jj)

    def body(p, carry):
        not_last = p < NPAIR - 1
        for jj in range(16):
            half, pos, b = (jj // 8) % 2, jj % 8, jj % NBUF
            nhalf, npos = ((jj + 4) // 8) % 2, (jj + 4) % 8
            if jj == 4:
                _wait_window()
            if jj == 12:

                @pl.when(not_last)
                def _():
                    _wait_window()

            if jj < 12:
                _wait_gather(b)
                _scatter(half, pos, b)
                _gather(nhalf, npos, b)
            else:
                @pl.when(not_last)
                def _():
                    _wait_gather(b)
                    _scatter(half, pos, b)
                    _gather(nhalf, npos, b)

            if jj == 7:

                @pl.when(not_last)
                def _():
                    _load_window(2 * p + 2, 0, False)

            if jj == 15:

                @pl.when(not_last)
                def _():
                    _load_window(2 * p + 3, 1, False)
        return carry

    lax.fori_loop(0, NPAIR, body, 0)

    for jj in range(12, 16):
        b = jj % NBUF
        _wait_gather(b)
        _scatter(1, jj % 8, b)

    plsc.subcore_barrier()
    pltpu.sync_copy(agg_sh.at[pl.ds(r0, ROWS_PER_TILE)],
                    out_hbm.at[cid, pl.ds(r0, ROWS_PER_TILE)])


def _tc_body(data_ref, p0_ref, p1_ref, wr_ref, wn_ref, b_ref, gam_ref,
             bet_ref, g_ref, gt_ref, out_ref):
    x = data_ref[...]
    agg = p0_ref[...] + p1_ref[...]
    acc = jnp.dot(x, wr_ref[...], preferred_element_type=jnp.float32)
    acc = acc + jnp.dot(agg, wn_ref[...], preferred_element_type=jnp.float32)
    acc = acc + b_ref[...]
    g_mat = g_ref[...]
    gt_mat = gt_ref[...]
    m = jnp.dot(jnp.dot(acc, g_mat, preferred_element_type=jnp.float32),
                gt_mat, preferred_element_type=jnp.float32)
    e2 = jnp.dot(jnp.dot(acc * acc, g_mat, preferred_element_type=jnp.float32),
                 gt_mat, preferred_element_type=jnp.float32)
    var = e2 - m * m
    inv = lax.rsqrt(var + BN_EPS)
    out_ref[...] = (acc - m) * inv * gam_ref[...] + bet_ref[...]


def kernel(data, edge_index, depth, W_root, W_neigh, b, gamma, beta):
    del depth
    src = edge_index[0].astype(jnp.int32)
    dst = edge_index[1].astype(jnp.int32)
    pad = E_PAD - E
    pad_ar = jnp.arange(pad, dtype=jnp.int32)
    src_p = jnp.concatenate([src, pad_ar % N_NODES])
    src_p = src_p.reshape(NW, 2 * NPAIR, WCHUNKS, CHUNK)
    dst_p = jnp.concatenate([dst, N_NODES + pad_ar % (AGG_ROWS - N_NODES)])
    dst_p = dst_p.reshape(NW, 2 * NPAIR, WCHUNKS, CHUNK)
    zeros = jnp.zeros((AGG_ROWS, D), jnp.float32)

    partials = _sc_agg(data, zeros, src_p, dst_p)

    gsz = D // NUM_GROUPS
    ch = jnp.arange(D, dtype=jnp.int32) // gsz
    gr = jnp.arange(8, dtype=jnp.int32)
    g_mat = (ch[:, None] == gr[None, :]).astype(jnp.float32) / gsz
    gt_mat = (gr[:, None] == ch[None, :]).astype(jnp.float32)

    blk = 1000
    grid = (N_NODES // blk,)
    row_spec = pl.BlockSpec((blk, D), lambda i: (i, 0))
    full = lambda r, c: pl.BlockSpec((r, c), lambda i: (0, 0))
    out = pl.pallas_call(
        _tc_body,
        grid=grid,
        in_specs=[
            row_spec,
            row_spec,
            row_spec,
            full(D, D),
            full(D, D),
            full(1, D),
            full(1, D),
            full(1, D),
            full(D, 8),
            full(8, D),
        ],
        out_specs=row_spec,
        out_shape=jax.ShapeDtypeStruct((N_NODES, D), jnp.float32),
    )(data, partials[0], partials[1], W_root, W_neigh,
      b.reshape(1, D), gamma.reshape(1, D), beta.reshape(1, D), g_mat, gt_mat)
    return out


# speedup vs baseline: 1.0058x; 1.0005x over previous
import functools

import jax
import jax.numpy as jnp
from jax import lax
from jax.experimental import pallas as pl
from jax.experimental.pallas import tpu as pltpu
from jax.experimental.pallas import tpu_sc as plsc

N_NODES = 10000
D = 128
E = 320000
NUM_GROUPS = 4
BN_EPS = 1e-5

NC = 2
NS = 16
NW = NC * NS

CHUNK = 80
NBUF = 4
WCHUNKS = 8
NPAIR = 8
CHUNKS_PER_TILE = 2 * WCHUNKS * NPAIR
EDGES_PER_TILE = CHUNK * CHUNKS_PER_TILE
E_PAD = EDGES_PER_TILE * NW
AGG_ROWS = 10112
ROWS_PER_TILE = AGG_ROWS // NS

_mesh = plsc.VectorSubcoreMesh(core_axis_name="c", subcore_axis_name="s")


@functools.partial(
    pl.kernel,
    out_type=jax.ShapeDtypeStruct((NC, AGG_ROWS, D), jnp.float32),
    mesh=_mesh,
    scratch_types=[
        pltpu.VMEM((2, 2, WCHUNKS, CHUNK), jnp.int32),
        pltpu.VMEM((NBUF, CHUNK, D), jnp.float32),
        pltpu.VMEM_SHARED((AGG_ROWS, D), jnp.float32),
        pltpu.SemaphoreType.DMA,
        pltpu.SemaphoreType.DMA,
        pltpu.SemaphoreType.DMA,
        pltpu.SemaphoreType.DMA,
        pltpu.SemaphoreType.DMA,
    ],
)
def _sc_agg(data_hbm, zeros_hbm, src_hbm, dst_hbm, out_hbm,
            idx_w, rows_v, agg_sh, sem0, sem1, sem2, sem3, sem_i):
    cid = lax.axis_index("c")
    sid = lax.axis_index("s")
    wid = sid * NC + cid
    r0 = sid * ROWS_PER_TILE
    sems = (sem0, sem1, sem2, sem3)

    pltpu.sync_copy(zeros_hbm.at[pl.ds(r0, ROWS_PER_TILE)],
                    agg_sh.at[pl.ds(r0, ROWS_PER_TILE)])

    def _load_window(w, half, sync):
        pltpu.async_copy(src_hbm.at[wid, w], idx_w.at[half, 0], sem_i)
        cp = pltpu.async_copy(dst_hbm.at[wid, w], idx_w.at[half, 1], sem_i)
        if sync:
            _wait_window()

    def _wait_window():
        pltpu.make_async_copy(src_hbm.at[0, 0], idx_w.at[0, 0], sem_i).wait()
        pltpu.make_async_copy(dst_hbm.at[0, 0], idx_w.at[0, 1], sem_i).wait()

    H = CHUNK // 2

    def _gather(half, pos, b):
        for h2 in (0, H):
            pltpu.async_copy(data_hbm.at[idx_w.at[half, 0, pos, pl.ds(h2, H)]],
                             rows_v.at[b, pl.ds(h2, H)], sems[b])

    def _wait_gather(b):
        for h2 in (0, H):
            pltpu.make_async_copy(data_hbm.at[pl.ds(0, H)],
                                  rows_v.at[b, pl.ds(h2, H)], sems[b]).wait()

    def _scatter(half, pos, b):
        pltpu.sync_copy(rows_v.at[b], agg_sh.at[idx_w.at[half, 1, pos]],
                        add=True)

    _load_window(0, 0, True)
    plsc.subcore_barrier()
    _load_window(1, 1, False)
    for jj in range(NBUF):
        _gather(0, jj, jj)

    def body(p, carry):
        not_last = p < NPAIR - 1
        for jj in range(16):
            half, pos, b = (jj // 8) % 2, jj % 8, jj % NBUF
            nhalf, npos = ((jj + 4) // 8) % 2, (jj + 4) % 8
            if jj == 4:
                _wait_window()
            if jj == 12:

                @pl.when(not_last)
                def _():
                    _wait_window()

            if jj < 12:
                _wait_gather(b)
                _scatter(half, pos, b)
                _gather(nhalf, npos, b)
            else:
                @pl.when(not_last)
                def _():
                    _wait_gather(b)
                    _scatter(half, pos, b)
                    _gather(nhalf, npos, b)

            if jj == 7:

                @pl.when(not_last)
                def _():
                    _load_window(2 * p + 2, 0, False)

            if jj == 15:

                @pl.when(not_last)
                def _():
                    _load_window(2 * p + 3, 1, False)
        return carry

    lax.fori_loop(0, NPAIR, body, 0)

    for jj in range(12, 16):
        b = jj % NBUF
        _wait_gather(b)
        _scatter(1, jj % 8, b)

    plsc.subcore_barrier()
    pltpu.sync_copy(agg_sh.at[pl.ds(r0, ROWS_PER_TILE)],
                    out_hbm.at[cid, pl.ds(r0, ROWS_PER_TILE)])


def _tc_body(data_ref, p0_ref, p1_ref, wr_ref, wn_ref, b_ref, gam_ref,
             bet_ref, g_ref, gt_ref, out_ref):
    x = data_ref[...]
    agg = p0_ref[...] + p1_ref[...]
    acc = jnp.dot(x, wr_ref[...], preferred_element_type=jnp.float32)
    acc = acc + jnp.dot(agg, wn_ref[...], preferred_element_type=jnp.float32)
    acc = acc + b_ref[...]
    g_mat = g_ref[...]
    gt_mat = gt_ref[...]
    m = jnp.dot(jnp.dot(acc, g_mat, preferred_element_type=jnp.float32),
                gt_mat, preferred_element_type=jnp.float32)
    e2 = jnp.dot(jnp.dot(acc * acc, g_mat, preferred_element_type=jnp.float32),
                 gt_mat, preferred_element_type=jnp.float32)
    var = e2 - m * m
    inv = lax.rsqrt(var + BN_EPS)
    out_ref[...] = (acc - m) * inv * gam_ref[...] + bet_ref[...]


def kernel(data, edge_index, depth, W_root, W_neigh, b, gamma, beta):
    del depth
    src = edge_index[0].astype(jnp.int32)
    dst = edge_index[1].astype(jnp.int32)
    pad = E_PAD - E
    pad_ar = jnp.arange(pad, dtype=jnp.int32)
    src_p = jnp.concatenate([src, pad_ar % N_NODES])
    src_p = src_p.reshape(NW, 2 * NPAIR, WCHUNKS, CHUNK)
    dst_p = jnp.concatenate([dst, N_NODES + pad_ar % (AGG_ROWS - N_NODES)])
    dst_p = dst_p.reshape(NW, 2 * NPAIR, WCHUNKS, CHUNK)
    zeros = jnp.zeros((AGG_ROWS, D), jnp.float32)

    partials = _sc_agg(data, zeros, src_p, dst_p)

    gsz = D // NUM_GROUPS
    ch = jnp.arange(D, dtype=jnp.int32) // gsz
    gr = jnp.arange(8, dtype=jnp.int32)
    g_mat = (ch[:, None] == gr[None, :]).astype(jnp.float32) / gsz
    gt_mat = (gr[:, None] == ch[None, :]).astype(jnp.float32)

    blk = 1000
    grid = (N_NODES // blk,)
    row_spec = pl.BlockSpec((blk, D), lambda i: (i, 0))
    full = lambda r, c: pl.BlockSpec((r, c), lambda i: (0, 0))
    out = pl.pallas_call(
        _tc_body,
        grid=grid,
        in_specs=[
            row_spec,
            row_spec,
            row_spec,
            full(D, D),
            full(D, D),
            full(1, D),
            full(1, D),
            full(1, D),
            full(D, 8),
            full(8, D),
        ],
        out_specs=row_spec,
        out_shape=jax.ShapeDtypeStruct((N_NODES, D), jnp.float32),
    )(data, partials[0], partials[1], W_root, W_neigh,
      b.reshape(1, D), gamma.reshape(1, D), beta.reshape(1, D), g_mat, gt_mat)
    return out


# speedup vs baseline: 1.0128x; 1.0070x over previous
import functools

import jax
import jax.numpy as jnp
from jax import lax
from jax.experimental import pallas as pl
from jax.experimental.pallas import tpu as pltpu
from jax.experimental.pallas import tpu_sc as plsc

N_NODES = 10000
D = 128
E = 320000
NUM_GROUPS = 4
BN_EPS = 1e-5

NC = 2
NS = 16
NW = NC * NS

CHUNK = 80
NBUF = 4
WCHUNKS = 8
NPAIR = 8
CHUNKS_PER_TILE = 2 * WCHUNKS * NPAIR
EDGES_PER_TILE = CHUNK * CHUNKS_PER_TILE
E_PAD = EDGES_PER_TILE * NW
AGG_ROWS = 10112
ROWS_PER_TILE = AGG_ROWS // NS

_mesh = plsc.VectorSubcoreMesh(core_axis_name="c", subcore_axis_name="s")


@functools.partial(
    pl.kernel,
    out_type=jax.ShapeDtypeStruct((NC, AGG_ROWS, D), jnp.float32),
    mesh=_mesh,
    scratch_types=[
        pltpu.VMEM((2, 2, WCHUNKS, CHUNK), jnp.int32),
        pltpu.VMEM((NBUF, CHUNK, D), jnp.float32),
        pltpu.VMEM_SHARED((AGG_ROWS, D), jnp.float32),
        pltpu.SemaphoreType.DMA,
        pltpu.SemaphoreType.DMA,
        pltpu.SemaphoreType.DMA,
        pltpu.SemaphoreType.DMA,
        pltpu.SemaphoreType.DMA,
    ],
)
def _sc_agg(data_hbm, zeros_hbm, src_hbm, dst_hbm, out_hbm,
            idx_w, rows_v, agg_sh, sem0, sem1, sem2, sem3, sem_i):
    cid = lax.axis_index("c")
    sid = lax.axis_index("s")
    wid = sid * NC + cid
    r0 = sid * ROWS_PER_TILE
    sems = (sem0, sem1, sem2, sem3)

    pltpu.sync_copy(zeros_hbm, agg_sh.at[pl.ds(r0, ROWS_PER_TILE)])

    def _load_window(w, half, sync):
        pltpu.async_copy(src_hbm.at[wid, w], idx_w.at[half, 0], sem_i)
        cp = pltpu.async_copy(dst_hbm.at[wid, w], idx_w.at[half, 1], sem_i)
        if sync:
            _wait_window()

    def _wait_window():
        pltpu.make_async_copy(src_hbm.at[0, 0], idx_w.at[0, 0], sem_i).wait()
        pltpu.make_async_copy(dst_hbm.at[0, 0], idx_w.at[0, 1], sem_i).wait()

    def _gather(half, pos, b):
        pltpu.async_copy(data_hbm.at[idx_w.at[half, 0, pos]],
                         rows_v.at[b], sems[b])

    def _wait_gather(b):
        pltpu.make_async_copy(data_hbm.at[pl.ds(0, CHUNK)], rows_v.at[b],
                              sems[b]).wait()

    def _scatter(half, pos, b):
        pltpu.sync_copy(rows_v.at[b], agg_sh.at[idx_w.at[half, 1, pos]],
                        add=True)

    _load_window(0, 0, True)
    plsc.subcore_barrier()
    _load_window(1, 1, False)
    for jj in range(NBUF):
        _gather(0, jj, jj)

    def body(p, carry):
        not_last = p < NPAIR - 1
        for jj in range(16):
            half, pos, b = (jj // 8) % 2, jj % 8, jj % NBUF
            nhalf, npos = ((jj + 4) // 8) % 2, (jj + 4) % 8
            if jj == 4:
                _wait_window()
            if jj == 12:

                @pl.when(not_last)
                def _():
                    _wait_window()

            if jj < 12:
                _wait_gather(b)
                _scatter(half, pos, b)
                _gather(nhalf, npos, b)
            else:
                @pl.when(not_last)
                def _():
                    _wait_gather(b)
                    _scatter(half, pos, b)
                    _gather(nhalf, npos, b)

            if jj == 7:

                @pl.when(not_last)
                def _():
                    _load_window(2 * p + 2, 0, False)

            if jj == 15:

                @pl.when(not_last)
                def _():
                    _load_window(2 * p + 3, 1, False)
        return carry

    lax.fori_loop(0, NPAIR, body, 0)

    for jj in range(12, 16):
        b = jj % NBUF
        _wait_gather(b)
        _scatter(1, jj % 8, b)

    plsc.subcore_barrier()
    pltpu.sync_copy(agg_sh.at[pl.ds(r0, ROWS_PER_TILE)],
                    out_hbm.at[cid, pl.ds(r0, ROWS_PER_TILE)])


def _tc_body(data_ref, p0_ref, p1_ref, wr_ref, wn_ref, b_ref, gam_ref,
             bet_ref, g_ref, gt_ref, out_ref):
    x = data_ref[...]
    agg = p0_ref[...] + p1_ref[...]
    acc = jnp.dot(x, wr_ref[...], preferred_element_type=jnp.float32)
    acc = acc + jnp.dot(agg, wn_ref[...], preferred_element_type=jnp.float32)
    acc = acc + b_ref[...]
    g_mat = g_ref[...]
    gt_mat = gt_ref[...]
    m = jnp.dot(jnp.dot(acc, g_mat, preferred_element_type=jnp.float32),
                gt_mat, preferred_element_type=jnp.float32)
    e2 = jnp.dot(jnp.dot(acc * acc, g_mat, preferred_element_type=jnp.float32),
                 gt_mat, preferred_element_type=jnp.float32)
    var = e2 - m * m
    inv = lax.rsqrt(var + BN_EPS)
    out_ref[...] = (acc - m) * inv * gam_ref[...] + bet_ref[...]


def kernel(data, edge_index, depth, W_root, W_neigh, b, gamma, beta):
    del depth
    src = edge_index[0].astype(jnp.int32)
    dst = edge_index[1].astype(jnp.int32)
    pad = E_PAD - E
    pad_ar = jnp.arange(pad, dtype=jnp.int32)
    src_p = jnp.concatenate([src, pad_ar % N_NODES])
    src_p = src_p.reshape(NW, 2 * NPAIR, WCHUNKS, CHUNK)
    dst_p = jnp.concatenate([dst, N_NODES + pad_ar % (AGG_ROWS - N_NODES)])
    dst_p = dst_p.reshape(NW, 2 * NPAIR, WCHUNKS, CHUNK)
    zeros = jnp.zeros((ROWS_PER_TILE, D), jnp.float32)

    partials = _sc_agg(data, zeros, src_p, dst_p)

    gsz = D // NUM_GROUPS
    ch = jnp.arange(D, dtype=jnp.int32) // gsz
    gr = jnp.arange(8, dtype=jnp.int32)
    g_mat = (ch[:, None] == gr[None, :]).astype(jnp.float32) / gsz
    gt_mat = (gr[:, None] == ch[None, :]).astype(jnp.float32)

    blk = 1000
    grid = (N_NODES // blk,)
    row_spec = pl.BlockSpec((blk, D), lambda i: (i, 0))
    full = lambda r, c: pl.BlockSpec((r, c), lambda i: (0, 0))
    out = pl.pallas_call(
        _tc_body,
        grid=grid,
        in_specs=[
            row_spec,
            row_spec,
            row_spec,
            full(D, D),
            full(D, D),
            full(1, D),
            full(1, D),
            full(1, D),
            full(D, 8),
            full(8, D),
        ],
        out_specs=row_spec,
        out_shape=jax.ShapeDtypeStruct((N_NODES, D), jnp.float32),
    )(data, partials[0], partials[1], W_root, W_neigh,
      b.reshape(1, D), gamma.reshape(1, D), beta.reshape(1, D), g_mat, gt_mat)
    return out
